# speedup vs baseline: 18.4009x; 18.4009x over previous
import jax
import jax.numpy as jnp
from jax.experimental import pallas as pl

N = 8192
STATE_DIM = 512
HID = 128
D_EMB = 64
KSEL = 33
EPS = 0.001
C = 0.001

_ROWS_A = 1024
_ROWS_B = 256

_INTERPRET = False


def _embed_body(s_ref, w1t_ref, b1_ref, w2t_ref, b2_ref, x_ref, xt_ref):
    h = jnp.dot(s_ref[...], w1t_ref[...], preferred_element_type=jnp.float32)
    h = jnp.maximum(h + b1_ref[...], 0.0)
    x = jnp.dot(h, w2t_ref[...], preferred_element_type=jnp.float32)
    x = x + b2_ref[...]
    x_ref[...] = x
    xt_ref[...] = x.T


def _knn_body(x_ref, xt_ref, out_ref):
    x_r = x_ref[...]
    xt = xt_ref[...]
    sq_r = jnp.sum(x_r * x_r, axis=1, keepdims=True)
    sq_c = jnp.sum(xt * xt, axis=0, keepdims=True)
    d2 = sq_r + sq_c - 2.0 * jnp.dot(x_r, xt, preferred_element_type=jnp.float32)
    d2 = jnp.maximum(d2, 1e-12)
    bits = jax.lax.bitcast_convert_type(d2, jnp.int32)

    rows = d2.shape[0]
    lo = jnp.zeros((rows, 1), jnp.int32)
    hi = jnp.max(bits, axis=1, keepdims=True)
    cnt_lo = jnp.zeros((rows, 1), jnp.int32)

    def step(_, carry):
        lo, hi, cnt_lo = carry
        mid = lo + ((hi - lo) >> 1)
        cnt = jnp.sum((bits <= mid).astype(jnp.int32), axis=1, keepdims=True)
        take_hi = cnt >= KSEL
        hi = jnp.where(take_hi, mid, hi)
        lo = jnp.where(take_hi, lo, mid)
        cnt_lo = jnp.where(take_hi, cnt_lo, cnt)
        return lo, hi, cnt_lo

    lo, hi, cnt_lo = jax.lax.fori_loop(0, 31, step, (lo, hi, cnt_lo))
    dist = jnp.sqrt(d2)
    tstar = jnp.sqrt(jax.lax.bitcast_convert_type(hi, jnp.float32))
    sum_lt = jnp.sum(jnp.where(bits <= lo, dist, 0.0), axis=1, keepdims=True)
    rowmin = jnp.sqrt(jnp.min(d2, axis=1, keepdims=True))
    out_ref[...] = (sum_lt + (KSEL - cnt_lo).astype(jnp.float32) * tstar
                    - rowmin)


def _epilogue_body(sk_ref, r_ref):
    sk = sk_ref[...]
    m2 = jnp.mean(sk) ** 2
    knn = EPS / (sk * sk / m2 + EPS)
    r_ref[...] = 1.0 / (jnp.sqrt(knn) + C)


def kernel(s, W1, b1, W2, b2):
    w1t = W1.T
    w2t = W2.T
    b1r = b1.reshape(1, HID)
    b2r = b2.reshape(1, D_EMB)

    x, xt = pl.pallas_call(
        _embed_body,
        grid=(N // _ROWS_A,),
        in_specs=[
            pl.BlockSpec((_ROWS_A, STATE_DIM), lambda i: (i, 0)),
            pl.BlockSpec((STATE_DIM, HID), lambda i: (0, 0)),
            pl.BlockSpec((1, HID), lambda i: (0, 0)),
            pl.BlockSpec((HID, D_EMB), lambda i: (0, 0)),
            pl.BlockSpec((1, D_EMB), lambda i: (0, 0)),
        ],
        out_specs=[
            pl.BlockSpec((_ROWS_A, D_EMB), lambda i: (i, 0)),
            pl.BlockSpec((D_EMB, _ROWS_A), lambda i: (0, i)),
        ],
        out_shape=[
            jax.ShapeDtypeStruct((N, D_EMB), jnp.float32),
            jax.ShapeDtypeStruct((D_EMB, N), jnp.float32),
        ],
        interpret=_INTERPRET,
    )(s, w1t, b1r, w2t, b2r)

    sum_k = pl.pallas_call(
        _knn_body,
        grid=(N // _ROWS_B,),
        in_specs=[
            pl.BlockSpec((_ROWS_B, D_EMB), lambda i: (i, 0)),
            pl.BlockSpec((D_EMB, N), lambda i: (0, 0)),
        ],
        out_specs=pl.BlockSpec((_ROWS_B, 1), lambda i: (i, 0)),
        out_shape=jax.ShapeDtypeStruct((N, 1), jnp.float32),
        interpret=_INTERPRET,
    )(x, xt)

    r = pl.pallas_call(
        _epilogue_body,
        out_shape=jax.ShapeDtypeStruct((N // 128, 128), jnp.float32),
        interpret=_INTERPRET,
    )(sum_k.reshape(N // 128, 128))

    return r.reshape(N, 1)


# speedup vs baseline: 19.5253x; 1.0611x over previous
import jax
import jax.numpy as jnp
from jax.experimental import pallas as pl

N = 8192
STATE_DIM = 512
HID = 128
D_EMB = 64
KSEL = 33
EPS = 0.001
C = 0.001

_ROWS_A = 1024
_ROWS_B = 256

_INTERPRET = False


def _embed_body(s_ref, w1t_ref, b1_ref, w2t_ref, b2_ref, x_ref, xt_ref):
    h = jnp.dot(s_ref[...], w1t_ref[...], preferred_element_type=jnp.float32)
    h = jnp.maximum(h + b1_ref[...], 0.0)
    x = jnp.dot(h, w2t_ref[...], preferred_element_type=jnp.float32)
    x = x + b2_ref[...]
    x_ref[...] = x
    xt_ref[...] = x.T


def _knn_body(x_ref, xt_ref, out_ref):
    x_r = x_ref[...]
    xt = xt_ref[...]
    sq_r = jnp.sum(x_r * x_r, axis=1, keepdims=True)
    sq_c = jnp.sum(xt * xt, axis=0, keepdims=True)
    d2 = sq_r + sq_c - 2.0 * jnp.dot(x_r, xt, preferred_element_type=jnp.float32)
    d2 = jnp.maximum(d2, 1e-12)
    bits = jax.lax.bitcast_convert_type(d2, jnp.int32)
    b16 = bits >> 16
    minbits = jnp.min(bits, axis=1, keepdims=True)
    lo = (minbits >> 16) - 1
    hi = jnp.max(bits, axis=1, keepdims=True) >> 16
    rows = d2.shape[0]
    cnt_lo = jnp.zeros((rows, 1), jnp.int32)

    def bs_cond(carry):
        lo, hi, cnt_lo = carry
        return jnp.any(hi - lo > 1)

    def bs_body(carry):
        lo, hi, cnt_lo = carry
        mid = lo + ((hi - lo) >> 1)
        cnt = jnp.sum((b16 <= mid).astype(jnp.int32), axis=1, keepdims=True)
        take_hi = cnt >= KSEL
        hi = jnp.where(take_hi, mid, hi)
        lo = jnp.where(take_hi, lo, mid)
        cnt_lo = jnp.where(take_hi, cnt_lo, cnt)
        return lo, hi, cnt_lo

    lo, hi, cnt_lo = jax.lax.while_loop(bs_cond, bs_body, (lo, hi, cnt_lo))
    sum_lo = jnp.sum(jnp.where(b16 <= lo, jnp.sqrt(d2), 0.0),
                     axis=1, keepdims=True)
    in_bucket = b16 == hi
    need = KSEL - cnt_lo
    acc = jnp.zeros((rows, 1), jnp.float32)
    mprev = jnp.zeros((rows, 1), jnp.float32)

    def rf_cond(carry):
        need, _, _ = carry
        return jnp.any(need > 0)

    def rf_body(carry):
        need, acc, mprev = carry
        cand = jnp.where(in_bucket & (d2 > mprev), d2, jnp.inf)
        m = jnp.min(cand, axis=1, keepdims=True)
        c = jnp.sum((cand == m).astype(jnp.int32), axis=1, keepdims=True)
        take = jnp.minimum(c, need)
        acc = acc + jnp.where(take > 0, take.astype(jnp.float32) * jnp.sqrt(m),
                              0.0)
        return need - take, acc, m

    need, acc, _ = jax.lax.while_loop(rf_cond, rf_body, (need, acc, mprev))
    rowmin = jnp.sqrt(jax.lax.bitcast_convert_type(minbits, jnp.float32))
    out_ref[...] = sum_lo + acc - rowmin


def _epilogue_body(sk_ref, r_ref):
    sk = sk_ref[...]
    m2 = jnp.mean(sk) ** 2
    knn = EPS / (sk * sk / m2 + EPS)
    r_ref[...] = 1.0 / (jnp.sqrt(knn) + C)


def kernel(s, W1, b1, W2, b2):
    w1t = W1.T
    w2t = W2.T
    b1r = b1.reshape(1, HID)
    b2r = b2.reshape(1, D_EMB)

    x, xt = pl.pallas_call(
        _embed_body,
        grid=(N // _ROWS_A,),
        in_specs=[
            pl.BlockSpec((_ROWS_A, STATE_DIM), lambda i: (i, 0)),
            pl.BlockSpec((STATE_DIM, HID), lambda i: (0, 0)),
            pl.BlockSpec((1, HID), lambda i: (0, 0)),
            pl.BlockSpec((HID, D_EMB), lambda i: (0, 0)),
            pl.BlockSpec((1, D_EMB), lambda i: (0, 0)),
        ],
        out_specs=[
            pl.BlockSpec((_ROWS_A, D_EMB), lambda i: (i, 0)),
            pl.BlockSpec((D_EMB, _ROWS_A), lambda i: (0, i)),
        ],
        out_shape=[
            jax.ShapeDtypeStruct((N, D_EMB), jnp.float32),
            jax.ShapeDtypeStruct((D_EMB, N), jnp.float32),
        ],
        interpret=_INTERPRET,
    )(s, w1t, b1r, w2t, b2r)

    sum_k = pl.pallas_call(
        _knn_body,
        grid=(N // _ROWS_B,),
        in_specs=[
            pl.BlockSpec((_ROWS_B, D_EMB), lambda i: (i, 0)),
            pl.BlockSpec((D_EMB, N), lambda i: (0, 0)),
        ],
        out_specs=pl.BlockSpec((_ROWS_B, 1), lambda i: (i, 0)),
        out_shape=jax.ShapeDtypeStruct((N, 1), jnp.float32),
        interpret=_INTERPRET,
    )(x, xt)

    r = pl.pallas_call(
        _epilogue_body,
        out_shape=jax.ShapeDtypeStruct((N // 128, 128), jnp.float32),
        interpret=_INTERPRET,
    )(sum_k.reshape(N // 128, 128))

    return r.reshape(N, 1)
